# baseline (speedup 1.0000x reference)
import jax
import jax.numpy as jnp
from jax import lax
from jax.experimental import pallas as pl
from jax.experimental.pallas import tpu as pltpu
from jax.experimental.pallas import tpu_sc as plsc

B = 4096
S = 200
D = 64
R = B * S
NC, NS = 2, 16
NW = NC * NS
ROWS_W = R // NW
G = 100
CB = 2
CHUNK = CB * S
NG = CHUNK // G
NCHUNK = ROWS_W // CHUNK
IROWS_W = ROWS_W // G
LANES = 16
DSUB = D // LANES


def _body(x_ref, table_ref, pos_ref, out_ref, idx_v, buf_v, pos_v, sem):
    wid = lax.axis_index("s") * NC + lax.axis_index("c")
    pltpu.sync_copy(pos_ref, pos_v)
    base_row = wid * ROWS_W
    base_irow = wid * IROWS_W

    def chunk_body(c, carry):
        row0 = base_row + c * CHUNK
        irow0 = base_irow + c * NG
        pltpu.sync_copy(x_ref.at[pl.ds(irow0, NG)], idx_v)
        descs = [
            pltpu.async_copy(
                table_ref.at[idx_v.at[j]],
                buf_v.at[pl.ds(j * G, G)],
                sem,
            )
            for j in range(NG)
        ]
        for dsc in descs:
            dsc.wait()

        def s_body(s, carry2):
            for d in range(DSUB):
                pv = pos_v[s, pl.ds(d * LANES, LANES)]
                for b in range(CB):
                    plsc.addupdate(buf_v.at[b * S + s, pl.ds(d * LANES, LANES)], pv)
            return carry2

        lax.fori_loop(0, S, s_body, 0, unroll=2)
        pltpu.sync_copy(buf_v, out_ref.at[pl.ds(row0, CHUNK)])
        return carry

    lax.fori_loop(0, NCHUNK, chunk_body, 0)


import functools


@functools.lru_cache(maxsize=1)
def _make_gather_add():
    mesh = plsc.VectorSubcoreMesh(
        core_axis_name="c", subcore_axis_name="s", num_cores=NC, num_subcores=NS
    )
    return pl.kernel(
        _body,
        out_type=jax.ShapeDtypeStruct((R, D), jnp.float32),
        mesh=mesh,
        scratch_types=[
            pltpu.VMEM((NG, G), jnp.int32),
            pltpu.VMEM((CHUNK, D), jnp.float32),
            pltpu.VMEM((S, D), jnp.float32),
            pltpu.SemaphoreType.DMA,
        ],
        compiler_params=pltpu.CompilerParams(use_tc_tiling_on_sc=False),
    )


def kernel(x, table, pos_embed):
    x2d = x.reshape(R // G, G).astype(jnp.int32)
    pos2d = pos_embed.reshape(S, D).astype(jnp.float32)
    out = _make_gather_add()(x2d, table, pos2d)
    return out.reshape(B, S, D)
